# Initial kernel scaffold; baseline (speedup 1.0000x reference)
import jax
import jax.numpy as jnp
from jax.experimental import pallas as pl


def kernel(x, weight, bias, cheb_vals, cheb_rows, cheb_cols):
    raise NotImplementedError("write your pallas kernel here")


# speedup vs baseline: 9.1761x; 9.1761x over previous
import functools

import jax
import jax.numpy as jnp
from jax import lax
from jax.experimental import pallas as pl
from jax.experimental.pallas import tpu as pltpu
from jax.experimental.pallas import tpu_sc as plsc

_NNZ = 196608
_NW = 32
_PER_W = _NNZ // _NW
_K = 128
_GROUPS = _PER_W // _K
_BLK = 256
_NV = 4096
_NB = 12288


def _sc_spmm(x2b, vals_g, rows_g, cols_g):
    mesh = plsc.VectorSubcoreMesh(core_axis_name="core", subcore_axis_name="subcore")

    @functools.partial(
        pl.kernel,
        out_type=jax.ShapeDtypeStruct((2 * _NV, _BLK), jnp.float32),
        mesh=mesh,
        compiler_params=pltpu.CompilerParams(needs_layout_passes=False),
        scratch_types=[
            pltpu.VMEM((_GROUPS, _K), jnp.int32),
            pltpu.VMEM((_GROUPS, _K), jnp.int32),
            pltpu.VMEM((_GROUPS, _K), jnp.float32),
            pltpu.VMEM((_K, _BLK), jnp.float32),
        ],
    )
    def k(x_hbm, vals_hbm, rows_hbm, cols_hbm, z_hbm,
          cols_v, rows_v, vals_v, gbuf):
        cid = lax.axis_index("core")
        sid = lax.axis_index("subcore")
        wid = cid * 16 + sid

        pltpu.sync_copy(cols_hbm.at[wid], cols_v)
        pltpu.sync_copy(rows_hbm.at[wid], rows_v)
        pltpu.sync_copy(vals_hbm.at[wid], vals_v)

        roff = jnp.zeros((16,), jnp.int32) + cid * _NV

        @pl.loop(0, _GROUPS)
        def _(g):
            for h in range(_K // 16):
                sl = pl.ds(h * 16, 16)
                rows_v[g, sl] = rows_v[g, sl] + roff

        @pl.loop(0, _K)
        def _(r):
            for u in range(_BLK // 16):
                gbuf[r, pl.ds(u * 16, 16)] = jnp.zeros((16,), jnp.float32)

        pltpu.sync_copy(gbuf, z_hbm.at[pl.ds(wid * 256, _K)])
        pltpu.sync_copy(gbuf, z_hbm.at[pl.ds(wid * 256 + _K, _K)])
        plsc.subcore_barrier()

        @pl.loop(0, _GROUPS)
        def _(g):
            pltpu.sync_copy(x_hbm.at[cols_v.at[g]], gbuf)

            @pl.loop(0, _K)
            def _(e):
                idx_g = jnp.full((16,), g, jnp.int32)
                idx_e = jnp.full((16,), e, jnp.int32)
                val = plsc.load_gather(vals_v, [idx_g, idx_e])
                for u in range(_BLK // 16):
                    sl = pl.ds(u * 16, 16)
                    gbuf[e, sl] = gbuf[e, sl] * val

            pltpu.sync_copy(gbuf, z_hbm.at[rows_v.at[g]], add=True)

    return k(x2b, vals_g, rows_g, cols_g).reshape(2, _NV, _BLK)


def _tc_body(z_ref, bd_ref, b_ref, o_ref):
    zsum = z_ref[0] + z_ref[1]
    o_ref[...] = (
        jnp.dot(zsum, bd_ref[...], preferred_element_type=jnp.float32)
        + b_ref[...]
    )


def _tc_matmul(zp, bd, bias768):
    bm = 512
    return pl.pallas_call(
        _tc_body,
        grid=(_NV // bm,),
        in_specs=[
            pl.BlockSpec((2, bm, _BLK), lambda i: (0, i, 0)),
            pl.BlockSpec((_BLK, 768), lambda i: (0, 0)),
            pl.BlockSpec((1, 768), lambda i: (0, 0)),
        ],
        out_specs=pl.BlockSpec((bm, 768), lambda i: (i, 0)),
        out_shape=jax.ShapeDtypeStruct((_NV, 768), jnp.float32),
    )(zp, bd, bias768)


def kernel(x, weight, bias, cheb_vals, cheb_rows, cheb_cols):
    x2b = x.reshape(_NB, _BLK)
    cols_g = cheb_cols.reshape(_NW, _GROUPS, _K)
    rows_g = cheb_rows.reshape(_NW, _GROUPS, _K)
    vals_g = cheb_vals.reshape(_NW, _GROUPS, _K)

    zp = _sc_spmm(x2b, vals_g, rows_g, cols_g)

    wm = weight.reshape(32, 96)
    bd = (jnp.eye(8, dtype=jnp.float32)[:, None, :, None]
          * wm[None, :, None, :]).reshape(_BLK, 768)
    bias768 = jnp.tile(bias, 24).reshape(1, 768)

    out = _tc_matmul(zp, bd, bias768)
    return out.reshape(-1, 32)
